# Initial kernel scaffold; baseline (speedup 1.0000x reference)
import jax
import jax.numpy as jnp
from jax.experimental import pallas as pl


def kernel(features, tables, W1, b1, W2, b2):
    raise NotImplementedError("write your pallas kernel here")


# speedup vs baseline: 8.0447x; 8.0447x over previous
import functools

import jax
import jax.numpy as jnp
from jax import lax
from jax.experimental import pallas as pl
from jax.experimental.pallas import tpu as pltpu
from jax.experimental.pallas import tpu_sc as plsc

N_FIELDS = 26
VOCAB = 100000
EMBED_DIM = 32
BATCH = 16384
HIDDEN = 512
OUT = 128
INPUT_DIM = N_FIELDS * EMBED_DIM

NC = 2
NS = 16
NW = NC * NS

TOTAL_ROWS = BATCH * N_FIELDS
ROWS_PER_W = TOTAL_ROWS // NW
G = 128
NSTREAM = 8
GROUP = G * NSTREAM
NGROUPS = ROWS_PER_W // GROUP


def _sc_gather(features_flat, tables_flat):
    mesh = plsc.VectorSubcoreMesh(core_axis_name="c", subcore_axis_name="s")

    @functools.partial(
        pl.kernel,
        out_type=jax.ShapeDtypeStruct((TOTAL_ROWS, EMBED_DIM), jnp.float32),
        mesh=mesh,
        scratch_types=[
            pltpu.VMEM((ROWS_PER_W,), jnp.int32),
            pltpu.VMEM((ROWS_PER_W,), jnp.int32),
            [pltpu.VMEM((G, EMBED_DIM), jnp.float32) for _ in range(NSTREAM)],
            pltpu.SemaphoreType.DMA,
            pltpu.SemaphoreType.DMA,
        ],
        compiler_params=pltpu.CompilerParams(use_tc_tiling_on_sc=False),
    )
    def k(feat_hbm, tab_hbm, out_hbm, feat_v, idx_v, rows_v, gsem, wsem):
        wid = lax.axis_index("s") * NC + lax.axis_index("c")
        base = wid * ROWS_PER_W

        pltpu.sync_copy(feat_hbm.at[pl.ds(base, ROWS_PER_W)], feat_v)

        lane = lax.iota(jnp.int32, 16)

        def idx_body(i, _):
            r = i * 16
            field = lax.rem(r + lane, N_FIELDS)
            idx_v[pl.ds(r, 16)] = feat_v[pl.ds(r, 16)] + field * VOCAB
            return 0

        lax.fori_loop(0, ROWS_PER_W // 16, idx_body, 0)

        def group_body(g, _):
            row0 = g * GROUP
            gd = []
            for b in range(NSTREAM):
                gd.append(pltpu.async_copy(
                    tab_hbm.at[idx_v.at[pl.ds(row0 + b * G, G)]],
                    rows_v[b], gsem))
            for b in range(NSTREAM):
                gd[b].wait()
            wd = []
            for b in range(NSTREAM):
                wd.append(pltpu.async_copy(
                    rows_v[b],
                    out_hbm.at[pl.ds(base + row0 + b * G, G)], wsem))
            for b in range(NSTREAM):
                wd[b].wait()
            return 0

        lax.fori_loop(0, NGROUPS, group_body, 0)

    return k(features_flat, tables_flat)


def _mlp(x, W1, b1, W2, b2):
    BT = 1024
    grid = (BATCH // BT,)

    def body(x_ref, w1_ref, b1_ref, w2_ref, b2_ref, out_ref):
        h = jnp.dot(x_ref[...], w1_ref[...],
                    preferred_element_type=jnp.float32) + b1_ref[...]
        h = jnp.maximum(h, 0.0)
        out_ref[...] = jnp.dot(h, w2_ref[...],
                               preferred_element_type=jnp.float32) + b2_ref[...]

    return pl.pallas_call(
        body,
        grid=grid,
        in_specs=[
            pl.BlockSpec((BT, INPUT_DIM), lambda i: (i, 0)),
            pl.BlockSpec((INPUT_DIM, HIDDEN), lambda i: (0, 0)),
            pl.BlockSpec((1, HIDDEN), lambda i: (0, 0)),
            pl.BlockSpec((HIDDEN, OUT), lambda i: (0, 0)),
            pl.BlockSpec((1, OUT), lambda i: (0, 0)),
        ],
        out_specs=pl.BlockSpec((BT, OUT), lambda i: (i, 0)),
        out_shape=jax.ShapeDtypeStruct((BATCH, OUT), jnp.float32),
    )(x, W1, b1.reshape(1, HIDDEN), W2, b2.reshape(1, OUT))


def kernel(features, tables, W1, b1, W2, b2):
    features_flat = features.reshape(-1).astype(jnp.int32)
    tables_flat = tables.reshape(N_FIELDS * VOCAB, EMBED_DIM)
    rows = _sc_gather(features_flat, tables_flat)
    x = rows.reshape(BATCH, INPUT_DIM)
    return _mlp(x, W1, b1, W2, b2)


# speedup vs baseline: 11.7544x; 1.4611x over previous
import functools

import jax
import jax.numpy as jnp
from jax import lax
from jax.experimental import pallas as pl
from jax.experimental.pallas import tpu as pltpu
from jax.experimental.pallas import tpu_sc as plsc

N_FIELDS = 26
VOCAB = 100000
EMBED_DIM = 32
BATCH = 16384
HIDDEN = 512
OUT = 128
INPUT_DIM = N_FIELDS * EMBED_DIM

NC = 2
NS = 16
NW = NC * NS

PIECE = 4096
NPIECE = BATCH // PIECE


def _sc_gather_t(features_t, tables_em):
    mesh = plsc.VectorSubcoreMesh(core_axis_name="c", subcore_axis_name="s",
                                  num_cores=NC, num_subcores=NS)

    @functools.partial(
        pl.kernel,
        out_type=jax.ShapeDtypeStruct((INPUT_DIM, BATCH), jnp.float32),
        mesh=mesh,
        scratch_types=[
            pltpu.VMEM((VOCAB,), jnp.float32),
            pltpu.VMEM((BATCH,), jnp.int32),
            [pltpu.VMEM((PIECE,), jnp.float32) for _ in range(2)],
            pltpu.SemaphoreType.DMA,
        ],
        compiler_params=pltpu.CompilerParams(use_tc_tiling_on_sc=False,
                                             needs_layout_passes=False),
    )
    def k(feat_hbm, tab_hbm, out_hbm, vvec, feat_v, piece_v, wsem):
        e = lax.axis_index("s") * NC + lax.axis_index("c")

        for f in range(N_FIELDS):
            pltpu.sync_copy(tab_hbm.at[f, e], vvec)
            pltpu.sync_copy(feat_hbm.at[f], feat_v)
            row = f * EMBED_DIM + e
            wd = [None, None]
            for p in range(NPIECE):
                buf = piece_v[p % 2]
                if wd[p % 2] is not None:
                    wd[p % 2].wait()

                def gbody(i, _):
                    idx = feat_v[pl.ds(p * PIECE + i * 16, 16)]
                    buf[pl.ds(i * 16, 16)] = plsc.load_gather(vvec, [idx])
                    return 0

                lax.fori_loop(0, PIECE // 16, gbody, 0)
                wd[p % 2] = pltpu.async_copy(
                    buf, out_hbm.at[row, pl.ds(p * PIECE, PIECE)], wsem)
            for d in wd:
                if d is not None:
                    d.wait()

    return k(features_t, tables_em)


def _mlp_t(xt, W1, b1, W2, b2):
    BT = 2048
    grid = (BATCH // BT,)

    def body(xt_ref, w1_ref, b1_ref, w2_ref, b2_ref, out_ref):
        h = lax.dot_general(w1_ref[...], xt_ref[...],
                            (((0,), (0,)), ((), ())),
                            preferred_element_type=jnp.float32)
        h = jnp.maximum(h + b1_ref[...], 0.0)
        y = lax.dot_general(h, w2_ref[...],
                            (((0,), (0,)), ((), ())),
                            preferred_element_type=jnp.float32)
        out_ref[...] = y + b2_ref[...]

    return pl.pallas_call(
        body,
        grid=grid,
        in_specs=[
            pl.BlockSpec((INPUT_DIM, BT), lambda i: (0, i)),
            pl.BlockSpec((INPUT_DIM, HIDDEN), lambda i: (0, 0)),
            pl.BlockSpec((HIDDEN, 1), lambda i: (0, 0)),
            pl.BlockSpec((HIDDEN, OUT), lambda i: (0, 0)),
            pl.BlockSpec((1, OUT), lambda i: (0, 0)),
        ],
        out_specs=pl.BlockSpec((BT, OUT), lambda i: (i, 0)),
        out_shape=jax.ShapeDtypeStruct((BATCH, OUT), jnp.float32),
    )(xt, W1, b1.reshape(HIDDEN, 1), W2, b2.reshape(1, OUT))


def kernel(features, tables, W1, b1, W2, b2):
    features_t = features.T.astype(jnp.int32)
    tables_em = jnp.transpose(tables, (0, 2, 1))
    xt = _sc_gather_t(features_t, tables_em)
    return _mlp_t(xt, W1, b1, W2, b2)
